# Initial kernel scaffold; baseline (speedup 1.0000x reference)
import jax
import jax.numpy as jnp
from jax.experimental import pallas as pl


def kernel(cts, smlss, emb1, emb2, W1, b1, W2, b2):
    raise NotImplementedError("write your pallas kernel here")


# speedup vs baseline: 2.6171x; 2.6171x over previous
import functools

import jax
import jax.numpy as jnp
from jax import lax
from jax.experimental import pallas as pl
from jax.experimental.pallas import tpu as pltpu

B = 16384
N1 = 6
N2 = 146
EMB = 64
EMBED = 128
OUT = 128
BLK = 2048


def _argmax_first(x, n):
    blk = x.shape[0]
    iota = lax.broadcasted_iota(jnp.int32, (blk, n), 1)
    m = jnp.max(x, axis=1, keepdims=True)
    return jnp.min(jnp.where(x == m, iota, n), axis=1)


def _body(cts_ref, smlss_ref, emb1_ref, emb2_ref, w1_ref, b1_ref, w2_ref,
          b2_ref, out_ref):
    cts = cts_ref[...]
    smlss = smlss_ref[...]
    i1 = _argmax_first(cts, N1)
    i2 = _argmax_first(smlss, N2)
    oh1 = (lax.broadcasted_iota(jnp.int32, (BLK, N1), 1) == i1[:, None]
           ).astype(jnp.float32)
    oh2 = (lax.broadcasted_iota(jnp.int32, (BLK, N2), 1) == i2[:, None]
           ).astype(jnp.float32)
    t1 = jnp.dot(emb1_ref[...], w1_ref[:EMB, :],
                 preferred_element_type=jnp.float32)
    t2 = jnp.dot(emb2_ref[...], w1_ref[EMB:, :],
                 preferred_element_type=jnp.float32)
    h = (jnp.dot(oh1, t1, preferred_element_type=jnp.float32)
         + jnp.dot(oh2, t2, preferred_element_type=jnp.float32)
         + b1_ref[...])
    h = jnp.maximum(h, 0.0)
    out_ref[...] = (jnp.dot(h, w2_ref[...], preferred_element_type=jnp.float32)
                    + b2_ref[...])


@jax.jit
def _run(cts, smlss, emb1, emb2, W1, b1, W2, b2):
    grid = (B // BLK,)
    return pl.pallas_call(
        _body,
        grid=grid,
        in_specs=[
            pl.BlockSpec((BLK, N1), lambda i: (i, 0)),
            pl.BlockSpec((BLK, N2), lambda i: (i, 0)),
            pl.BlockSpec((N1, EMB), lambda i: (0, 0)),
            pl.BlockSpec((N2, EMB), lambda i: (0, 0)),
            pl.BlockSpec((EMBED, EMBED), lambda i: (0, 0)),
            pl.BlockSpec((1, EMBED), lambda i: (0, 0)),
            pl.BlockSpec((EMBED, OUT), lambda i: (0, 0)),
            pl.BlockSpec((1, OUT), lambda i: (0, 0)),
        ],
        out_specs=pl.BlockSpec((BLK, OUT), lambda i: (i, 0)),
        out_shape=jax.ShapeDtypeStruct((B, OUT), jnp.float32),
    )(cts, smlss, emb1, emb2, W1, b1.reshape(1, EMBED), W2,
      b2.reshape(1, OUT))


def kernel(cts, smlss, emb1, emb2, W1, b1, W2, b2):
    return _run(cts, smlss, emb1, emb2, W1, b1, W2, b2)
